# Initial kernel scaffold; baseline (speedup 1.0000x reference)
import jax
import jax.numpy as jnp
from jax import lax
from jax.experimental import pallas as pl
from jax.experimental.pallas import tpu as pltpu
from jax.experimental.pallas import tpu_sc as plsc

N_USERS = 5000
N_ITEMS = 5000
N = N_USERS + N_ITEMS
H = 128
E = 320000

NC = 2
NS = 16
NW = NC * NS
CHUNK = 128
CH = -(-E // (NW * CHUNK))
EP = NW * CH * CHUNK
ZROWS = 125
RPS = N // NS


def _spmm_body(x_hbm, src_hbm, dst_hbm, w_hbm, out_hbm,
               srcbuf, dstbuf, wbuf, rows, zbuf, accum, sem):
    c = lax.axis_index("c")
    s = lax.axis_index("s")
    wid = s * NC + c

    def zfill(i, carry):
        for g in range(H // 16):
            zbuf[i, pl.ds(g * 16, 16)] = jnp.zeros((16,), jnp.float32)
        return carry
    lax.fori_loop(0, ZROWS, zfill, 0)
    for k in range(RPS // ZROWS):
        pltpu.sync_copy(zbuf, accum.at[pl.ds(s * RPS + k * ZROWS, ZROWS)])
    plsc.subcore_barrier()

    def chunk_body(j, carry):
        pltpu.sync_copy(src_hbm.at[wid, j], srcbuf)
        pltpu.sync_copy(dst_hbm.at[wid, j], dstbuf)
        pltpu.sync_copy(w_hbm.at[wid, j], wbuf)
        pltpu.async_copy(x_hbm.at[srcbuf], rows, sem).wait()

        def scale(e, inner):
            w = wbuf[e]
            for g in range(H // 16):
                sl = pl.ds(g * 16, 16)
                rows[e, sl] = rows[e, sl] * w
            return inner
        lax.fori_loop(0, CHUNK, scale, 0)

        pltpu.sync_copy(rows, accum.at[dstbuf], add=True)
        return carry
    lax.fori_loop(0, CH, chunk_body, 0)

    plsc.subcore_barrier()
    base = s * RPS
    pltpu.sync_copy(accum.at[pl.ds(base, RPS)], out_hbm.at[c, pl.ds(base, RPS)])


_spmm = pl.kernel(
    _spmm_body,
    out_type=jax.ShapeDtypeStruct((NC, N, H), jnp.float32),
    mesh=plsc.VectorSubcoreMesh(core_axis_name="c", subcore_axis_name="s"),
    scratch_types=[
        pltpu.VMEM((CHUNK,), jnp.int32),
        pltpu.VMEM((CHUNK,), jnp.int32),
        pltpu.VMEM((CHUNK,), jnp.float32),
        pltpu.VMEM((CHUNK, H), jnp.float32),
        pltpu.VMEM((ZROWS, H), jnp.float32),
        pltpu.VMEM_SHARED((N, H), jnp.float32),
        pltpu.SemaphoreType.DMA,
    ],
)

_BLK = 1250


def _add2_body(a_ref, b_ref, o_ref):
    o_ref[...] = a_ref[...] + b_ref[...]


def _combine(p):
    return pl.pallas_call(
        _add2_body,
        out_shape=jax.ShapeDtypeStruct((N, H), jnp.float32),
        grid=(N // _BLK,),
        in_specs=[pl.BlockSpec((_BLK, H), lambda i: (i, 0)),
                  pl.BlockSpec((_BLK, H), lambda i: (i, 0))],
        out_specs=pl.BlockSpec((_BLK, H), lambda i: (i, 0)),
    )(p[0], p[1])


def _mean_body(e_ref, x1_ref, x2_ref, pa_ref, pb_ref, o_ref):
    o_ref[...] = 0.25 * (e_ref[...] + x1_ref[...] + x2_ref[...]
                         + pa_ref[...] + pb_ref[...])


def _mean(ego, x1, x2, pa, pb):
    spec = pl.BlockSpec((_BLK, H), lambda i: (i, 0))
    return pl.pallas_call(
        _mean_body,
        out_shape=jax.ShapeDtypeStruct((N, H), jnp.float32),
        grid=(N // _BLK,),
        in_specs=[spec] * 5,
        out_specs=spec,
    )(ego, x1, x2, pa, pb)


def kernel(adj_indices, adj_values, user_emb, item_emb):
    dst = adj_indices[0].astype(jnp.int32)
    src = adj_indices[1].astype(jnp.int32)
    w = adj_values.astype(jnp.float32)
    pad = EP - E
    src3 = jnp.pad(src, (0, pad)).reshape(NW, CH, CHUNK)
    dst3 = jnp.pad(dst, (0, pad)).reshape(NW, CH, CHUNK)
    w3 = jnp.pad(w, (0, pad)).reshape(NW, CH, CHUNK)

    ego = jnp.concatenate([user_emb, item_emb], axis=0)
    p1 = _spmm(ego, src3, dst3, w3)
    x1 = _combine(p1)
    p2 = _spmm(x1, src3, dst3, w3)
    x2 = _combine(p2)
    p3 = _spmm(x2, src3, dst3, w3)
    final = _mean(ego, x1, x2, p3[0], p3[1])
    return final[:N_USERS], final[N_USERS:]


# speedup vs baseline: 3.1151x; 3.1151x over previous
import jax
import jax.numpy as jnp
from jax import lax
from jax.experimental import pallas as pl
from jax.experimental.pallas import tpu as pltpu
from jax.experimental.pallas import tpu_sc as plsc

N_USERS = 5000
N_ITEMS = 5000
N = N_USERS + N_ITEMS
H = 128
E = 320000

NC = 2
NS = 16
NW = NC * NS
CHUNK = 128
CH = -(-E // (NW * CHUNK))
EP = NW * CH * CHUNK
NPAD = 10240
ZROWS = 128
RPS = NPAD // NS


def _spmm_body(x_hbm, src_hbm, dst_hbm, w_hbm, out_hbm,
               srcbuf, dstbuf, wbuf, rows, zbuf, accum, sem):
    c = lax.axis_index("c")
    s = lax.axis_index("s")
    wid = s * NC + c

    def zfill(i, carry):
        for g in range(H // 16):
            zbuf[i, pl.ds(g * 16, 16)] = jnp.zeros((16,), jnp.float32)
        return carry
    lax.fori_loop(0, ZROWS, zfill, 0)
    for k in range(RPS // ZROWS):
        pltpu.sync_copy(zbuf, accum.at[pl.ds(s * RPS + k * ZROWS, ZROWS)])
    plsc.subcore_barrier()

    def chunk_body(j, carry):
        pltpu.sync_copy(src_hbm.at[wid, j], srcbuf)
        pltpu.sync_copy(dst_hbm.at[wid, j], dstbuf)
        pltpu.sync_copy(w_hbm.at[wid, j], wbuf)
        pltpu.async_copy(x_hbm.at[srcbuf], rows, sem).wait()

        def scale(eg, inner):
            wv = wbuf[pl.ds(eg * 16, 16)]
            for i in range(16):
                w = wv[i]
                e = eg * 16 + i
                for g in range(H // 16):
                    sl = pl.ds(g * 16, 16)
                    rows[e, sl] = rows[e, sl] * w
            return inner
        lax.fori_loop(0, CHUNK // 16, scale, 0)

        pltpu.sync_copy(rows, accum.at[dstbuf], add=True)
        return carry
    lax.fori_loop(0, CH, chunk_body, 0)

    plsc.subcore_barrier()
    base = s * RPS
    pltpu.sync_copy(accum.at[pl.ds(base, RPS)], out_hbm.at[c, pl.ds(base, RPS)])


_spmm = pl.kernel(
    _spmm_body,
    out_type=jax.ShapeDtypeStruct((NC, NPAD, H), jnp.float32),
    mesh=plsc.VectorSubcoreMesh(core_axis_name="c", subcore_axis_name="s"),
    scratch_types=[
        pltpu.VMEM((CHUNK,), jnp.int32),
        pltpu.VMEM((CHUNK,), jnp.int32),
        pltpu.VMEM((CHUNK,), jnp.float32),
        pltpu.VMEM((CHUNK, H), jnp.float32),
        pltpu.VMEM((ZROWS, H), jnp.float32),
        pltpu.VMEM_SHARED((NPAD, H), jnp.float32),
        pltpu.SemaphoreType.DMA,
    ],
)

_BLK = 1024


def _add2_body(a_ref, b_ref, o_ref):
    o_ref[...] = a_ref[...] + b_ref[...]


def _combine(p):
    return pl.pallas_call(
        _add2_body,
        out_shape=jax.ShapeDtypeStruct((NPAD, H), jnp.float32),
        grid=(NPAD // _BLK,),
        in_specs=[pl.BlockSpec((_BLK, H), lambda i: (i, 0)),
                  pl.BlockSpec((_BLK, H), lambda i: (i, 0))],
        out_specs=pl.BlockSpec((_BLK, H), lambda i: (i, 0)),
    )(p[0], p[1])


def _mean_body(e_ref, x1_ref, x2_ref, pa_ref, pb_ref, o_ref):
    o_ref[...] = 0.25 * (e_ref[...] + x1_ref[...] + x2_ref[...]
                         + pa_ref[...] + pb_ref[...])


def _mean(ego, x1, x2, pa, pb):
    spec = pl.BlockSpec((_BLK, H), lambda i: (i, 0))
    return pl.pallas_call(
        _mean_body,
        out_shape=jax.ShapeDtypeStruct((NPAD, H), jnp.float32),
        grid=(NPAD // _BLK,),
        in_specs=[spec] * 5,
        out_specs=spec,
    )(ego, x1, x2, pa, pb)


def kernel(adj_indices, adj_values, user_emb, item_emb):
    dst = adj_indices[0].astype(jnp.int32)
    src = adj_indices[1].astype(jnp.int32)
    w = adj_values.astype(jnp.float32)
    pad = EP - E
    src3 = jnp.pad(src, (0, pad)).reshape(NW, CH, CHUNK)
    dst3 = jnp.pad(dst, (0, pad)).reshape(NW, CH, CHUNK)
    w3 = jnp.pad(w, (0, pad)).reshape(NW, CH, CHUNK)

    ego = jnp.pad(jnp.concatenate([user_emb, item_emb], axis=0),
                  ((0, NPAD - N), (0, 0)))
    p1 = _spmm(ego, src3, dst3, w3)
    x1 = _combine(p1)
    p2 = _spmm(x1, src3, dst3, w3)
    x2 = _combine(p2)
    p3 = _spmm(x2, src3, dst3, w3)
    final = _mean(ego, x1, x2, p3[0], p3[1])
    return final[:N_USERS], final[N_USERS:N]
